# baseline (speedup 1.0000x reference)
import jax
import jax.numpy as jnp
from jax.experimental import pallas as pl
from jax.experimental.pallas import tpu as pltpu

_ROWS = 8 * 8 * 32
_HD = 128
_NCHUNK = 2
_CH = _ROWS // _NCHUNK


def _copy_kernel(k_hbm, v_hbm, ko_hbm, vo_hbm,
                 kbuf, vbuf, ki_sems, ko_sems, vi_sems, vo_sems):
    for i in range(_NCHUNK):
        rows = pl.ds(i * _CH, _CH)
        pltpu.make_async_copy(k_hbm.at[rows], kbuf.at[i], ki_sems.at[i]).start()
        pltpu.make_async_copy(v_hbm.at[rows], vbuf.at[i], vi_sems.at[i]).start()
    for i in range(_NCHUNK):
        rows = pl.ds(i * _CH, _CH)
        pltpu.make_async_copy(k_hbm.at[rows], kbuf.at[i], ki_sems.at[i]).wait()
        pltpu.make_async_copy(kbuf.at[i], ko_hbm.at[rows], ko_sems.at[i]).start()
        pltpu.make_async_copy(v_hbm.at[rows], vbuf.at[i], vi_sems.at[i]).wait()
        pltpu.make_async_copy(vbuf.at[i], vo_hbm.at[rows], vo_sems.at[i]).start()
    for i in range(_NCHUNK):
        rows = pl.ds(i * _CH, _CH)
        pltpu.make_async_copy(kbuf.at[i], ko_hbm.at[rows], ko_sems.at[i]).wait()
        pltpu.make_async_copy(vbuf.at[i], vo_hbm.at[rows], vo_sems.at[i]).wait()


def kernel(key, value, key_cache, value_cache):
    del key_cache, value_cache
    out_shape = jax.ShapeDtypeStruct((_ROWS, _HD), key.dtype)
    ko, vo = pl.pallas_call(
        _copy_kernel,
        in_specs=[pl.BlockSpec(memory_space=pl.ANY)] * 2,
        out_specs=(pl.BlockSpec(memory_space=pl.ANY),) * 2,
        out_shape=(out_shape, out_shape),
        scratch_shapes=[
            pltpu.VMEM((_NCHUNK, _CH, _HD), jnp.float32),
            pltpu.VMEM((_NCHUNK, _CH, _HD), jnp.float32),
            pltpu.SemaphoreType.DMA((_NCHUNK,)),
            pltpu.SemaphoreType.DMA((_NCHUNK,)),
            pltpu.SemaphoreType.DMA((_NCHUNK,)),
            pltpu.SemaphoreType.DMA((_NCHUNK,)),
        ],
    )(key.reshape(_ROWS, _HD), value.reshape(_ROWS, _HD))
    return ko.reshape(key.shape), vo.reshape(value.shape)


# speedup vs baseline: 1.0035x; 1.0035x over previous
import jax
import jax.numpy as jnp
from jax.experimental import pallas as pl
from jax.experimental.pallas import tpu as pltpu

_ROWS = 8 * 8 * 32
_HD = 128
_NCHUNK = 4
_CH = _ROWS // _NCHUNK


def _copy_kernel(k_hbm, v_hbm, ko_hbm, vo_hbm,
                 kbuf, vbuf, ki_sems, ko_sems, vi_sems, vo_sems):
    for i in range(_NCHUNK):
        rows = pl.ds(i * _CH, _CH)
        pltpu.make_async_copy(k_hbm.at[rows], kbuf.at[i], ki_sems.at[i]).start()
        pltpu.make_async_copy(v_hbm.at[rows], vbuf.at[i], vi_sems.at[i]).start()
    for i in range(_NCHUNK):
        rows = pl.ds(i * _CH, _CH)
        pltpu.make_async_copy(k_hbm.at[rows], kbuf.at[i], ki_sems.at[i]).wait()
        pltpu.make_async_copy(kbuf.at[i], ko_hbm.at[rows], ko_sems.at[i]).start()
        pltpu.make_async_copy(v_hbm.at[rows], vbuf.at[i], vi_sems.at[i]).wait()
        pltpu.make_async_copy(vbuf.at[i], vo_hbm.at[rows], vo_sems.at[i]).start()
    for i in range(_NCHUNK):
        rows = pl.ds(i * _CH, _CH)
        pltpu.make_async_copy(kbuf.at[i], ko_hbm.at[rows], ko_sems.at[i]).wait()
        pltpu.make_async_copy(vbuf.at[i], vo_hbm.at[rows], vo_sems.at[i]).wait()


def kernel(key, value, key_cache, value_cache):
    del key_cache, value_cache
    out_shape = jax.ShapeDtypeStruct((_ROWS, _HD), key.dtype)
    ko, vo = pl.pallas_call(
        _copy_kernel,
        in_specs=[pl.BlockSpec(memory_space=pl.ANY)] * 2,
        out_specs=(pl.BlockSpec(memory_space=pl.ANY),) * 2,
        out_shape=(out_shape, out_shape),
        scratch_shapes=[
            pltpu.VMEM((_NCHUNK, _CH, _HD), jnp.float32),
            pltpu.VMEM((_NCHUNK, _CH, _HD), jnp.float32),
            pltpu.SemaphoreType.DMA((_NCHUNK,)),
            pltpu.SemaphoreType.DMA((_NCHUNK,)),
            pltpu.SemaphoreType.DMA((_NCHUNK,)),
            pltpu.SemaphoreType.DMA((_NCHUNK,)),
        ],
    )(key.reshape(_ROWS, _HD), value.reshape(_ROWS, _HD))
    return ko.reshape(key.shape), vo.reshape(value.shape)
